# baseline (speedup 1.0000x reference)
import functools

import jax
import jax.numpy as jnp
from jax import lax
from jax.experimental import pallas as pl
from jax.experimental.pallas import tpu as pltpu
from jax.experimental.pallas import tpu_sc as plsc

N = 10000
E = 320000
DF = 128
DE = 16
NHID = 128
NOUT = 128
NLAYER = 4
NGRAPH = 64
BN_EPS = 1e-5

NC = 2
NS = 16
CHUNK = 128
CPT = 160
IDXS = 40
PW = CPT * CHUNK
E_PAD = NS * PW
NHALF = N // NC
SENT = float(-1e9)

f32 = jnp.float32


def _dot16(a, w):
    return jnp.dot(a.astype(jnp.bfloat16), w.astype(jnp.bfloat16),
                   preferred_element_type=f32)


def _bn_cols(y, g, b):
    mu = jnp.mean(y, axis=0, keepdims=True)
    var = jnp.mean((y - mu) ** 2, axis=0, keepdims=True)
    return (y - mu) / jnp.sqrt(var + BN_EPS) * g + b


def _ea_stats_body(ea_ref, w_ref, s1_ref, s2_ref, acc1, acc2):
    j = pl.program_id(0)

    @pl.when(j == 0)
    def _():
        acc1[...] = jnp.zeros_like(acc1)
        acc2[...] = jnp.zeros_like(acc2)

    a = ea_ref[...]
    for l in range(NLAYER):
        d = _dot16(a, w_ref[l])
        acc1[l, ...] += jnp.sum(d, axis=0, keepdims=True)
        acc2[l, ...] += jnp.sum(d * d, axis=0, keepdims=True)

    @pl.when(j == pl.num_programs(0) - 1)
    def _():
        s1_ref[...] = acc1[...]
        s2_ref[...] = acc2[...]


_EA_BLK = 2048


def _ea_stats(ea_p, we):
    grid = E_PAD // _EA_BLK
    return pl.pallas_call(
        _ea_stats_body,
        grid=(grid,),
        in_specs=[pl.BlockSpec((_EA_BLK, DE), lambda j: (j, 0)),
                  pl.BlockSpec((NLAYER, DE, NHID), lambda j: (0, 0, 0))],
        out_specs=[pl.BlockSpec((NLAYER, 1, NHID), lambda j: (0, 0, 0)),
                   pl.BlockSpec((NLAYER, 1, NHID), lambda j: (0, 0, 0))],
        out_shape=[jax.ShapeDtypeStruct((NLAYER, 1, NHID), f32),
                   jax.ShapeDtypeStruct((NLAYER, 1, NHID), f32)],
        scratch_shapes=[pltpu.VMEM((NLAYER, 1, NHID), f32),
                        pltpu.VMEM((NLAYER, 1, NHID), f32)],
    )(ea_p, we)


def _ea_body(ea_ref, w_ref, off_ref, sig_ref, g_ref, t_ref, *out_refs):
    a = ea_ref[...]
    for l in range(NLAYER):
        y = (_dot16(a, w_ref[l]) + off_ref[l]) / sig_ref[l] * g_ref[l] + t_ref[l]
        out_refs[l][...] = jnp.maximum(y, 0.0)


_EA_BLK = 2048


def _edge_embeddings(ea_p, we, off, sig, g, t):
    grid = E_PAD // _EA_BLK
    return pl.pallas_call(
        _ea_body,
        grid=(grid,),
        in_specs=[pl.BlockSpec((_EA_BLK, DE), lambda j: (j, 0)),
                  pl.BlockSpec((NLAYER, DE, NHID), lambda j: (0, 0, 0)),
                  pl.BlockSpec((NLAYER, 1, NHID), lambda j: (0, 0, 0)),
                  pl.BlockSpec((NLAYER, 1, NHID), lambda j: (0, 0, 0)),
                  pl.BlockSpec((NLAYER, 1, NHID), lambda j: (0, 0, 0)),
                  pl.BlockSpec((NLAYER, 1, NHID), lambda j: (0, 0, 0))],
        out_specs=[pl.BlockSpec((_EA_BLK, NHID), lambda j: (j, 0))
                   for _ in range(NLAYER)],
        out_shape=[jax.ShapeDtypeStruct((E_PAD, NHID), f32)
                   for _ in range(NLAYER)],
    )(ea_p, we, off, sig, g, t)


def _enc_body(x_ref, w_ref, b_ref, g_ref, bb_ref, out_ref):
    y = _dot16(x_ref[...], w_ref[...]) + b_ref[...]
    out_ref[0:N, :] = jnp.maximum(_bn_cols(y, g_ref[...], bb_ref[...]), 0.0)
    out_ref[N:N + 8, :] = jnp.full((8, NHID), SENT, f32)


def _input_encoder(x, w, b, g, bb):
    return pl.pallas_call(
        _enc_body,
        out_shape=jax.ShapeDtypeStruct((N + 8, NHID), f32),
    )(x, w, b, g, bb)


def _node_body(h_ref, agg_ref, w_ref, g_ref, bb_ref, out_ref):
    h = h_ref[0:N, :]
    a = h + agg_ref[...]
    y = _dot16(a, w_ref[...])
    out_ref[0:N, :] = jnp.maximum(_bn_cols(y, g_ref[...], bb_ref[...]), 0.0) + h
    out_ref[N:N + 8, :] = jnp.full((8, NHID), SENT, f32)


def _node_update(h_s, agg, w, g, bb):
    return pl.pallas_call(
        _node_body,
        out_shape=jax.ShapeDtypeStruct((N + 8, NHID), f32),
    )(h_s, agg, w, g, bb)


def _pool_body(h_ref, bt_ref, w1, b1, g1, bb1, w2, b2, g2, bb2, out_ref):
    gid = lax.broadcasted_iota(jnp.int32, (NGRAPH, N), 0)
    onehot = (gid == bt_ref[...]).astype(f32)
    pooled = jnp.dot(onehot, h_ref[0:N, :], preferred_element_type=f32, precision=lax.Precision.HIGHEST)
    o = _dot16(pooled, w1[...]) + b1[...]
    o = jnp.maximum(_bn_cols(o, g1[...], bb1[...]), 0.0)
    o = _dot16(o, w2[...]) + b2[...]
    out_ref[...] = _bn_cols(o, g2[...], bb2[...])


def _pool_readout(h_s, batch_t, w1, b1, g1, bb1, w2, b2, g2, bb2):
    return pl.pallas_call(
        _pool_body,
        out_shape=jax.ShapeDtypeStruct((NGRAPH, NOUT), f32),
    )(h_s, batch_t, w1, b1, g1, bb1, w2, b2, g2, bb2)


def _sc_body(ea_hbm, src_hbm, dst_hbm, h_hbm, out_hbm,
             ea_v, g_v, src_v, dst_v, agg_sh, gsem):
    c = lax.axis_index("c")
    s = lax.axis_index("s")

    zero16 = jnp.zeros((16,), f32)

    def _zrow(i, carry):
        for cc in range(NHID // 16):
            g_v[i, pl.ds(cc * 16, 16)] = zero16
        return carry

    lax.fori_loop(0, CHUNK, _zrow, 0)

    base = s * 312

    @pl.when(s < NS - 1)
    def _():
        for k in range(2):
            pltpu.sync_copy(g_v, agg_sh.at[pl.ds(base + k * CHUNK, CHUNK)])
        pltpu.sync_copy(g_v.at[pl.ds(0, 56)],
                        agg_sh.at[pl.ds(base + 2 * CHUNK, 56)])

    @pl.when(s == NS - 1)
    def _():
        for k in range(2):
            pltpu.sync_copy(g_v, agg_sh.at[pl.ds(base + k * CHUNK, CHUNK)])
        pltpu.sync_copy(g_v.at[pl.ds(0, 64)],
                        agg_sh.at[pl.ds(base + 2 * CHUNK, 64)])

    plsc.subcore_barrier()

    for ph in range(CPT // IDXS):
        pltpu.sync_copy(src_hbm.at[c, s, pl.ds(ph * IDXS, IDXS)], src_v)
        pltpu.sync_copy(dst_hbm.at[c, s, pl.ds(ph * IDXS, IDXS)], dst_v)

        def _chunk(j, carry):
            jj = ph * IDXS + j
            pltpu.sync_copy(ea_hbm.at[s, pl.ds(jj * CHUNK, CHUNK)], ea_v)
            pltpu.async_copy(h_hbm.at[src_v.at[j]], g_v, gsem).wait()

            def _crow(i, cc2):
                for cc in range(NHID // 16):
                    sl = pl.ds(cc * 16, 16)
                    g_v[i, sl] = jnp.maximum(g_v[i, sl] + ea_v[i, sl], 0.0)
                return cc2

            lax.fori_loop(0, CHUNK, _crow, 0)
            pltpu.sync_copy(g_v, agg_sh.at[dst_v.at[j]], add=True)
            return carry

        lax.fori_loop(0, IDXS, _chunk, 0)
    plsc.subcore_barrier()

    @pl.when(s < NS - 1)
    def _():
        pltpu.sync_copy(agg_sh.at[pl.ds(base, 312)],
                        out_hbm.at[pl.ds(c * NHALF + base, 312)])

    @pl.when(s == NS - 1)
    def _():
        pltpu.sync_copy(agg_sh.at[pl.ds(base, 320)],
                        out_hbm.at[pl.ds(c * NHALF + base, 320)])


@functools.lru_cache(maxsize=1)
def _sc_scatter_kernel():
    return pl.kernel(
        _sc_body,
        out_type=jax.ShapeDtypeStruct((N, NHID), f32),
        mesh=plsc.VectorSubcoreMesh(core_axis_name="c", subcore_axis_name="s",
                                    num_cores=NC, num_subcores=NS),
        scratch_types=[
            pltpu.VMEM((CHUNK, NHID), f32),
            pltpu.VMEM((CHUNK, NHID), f32),
            pltpu.VMEM((IDXS, CHUNK), jnp.int32),
            pltpu.VMEM((IDXS, CHUNK), jnp.int32),
            pltpu.VMEM_SHARED((NHALF, NHID), f32),
            pltpu.SemaphoreType.DMA,
        ],
    )


def _sc_scatter(ea_l, src3, dst3, h_s):
    return _sc_scatter_kernel()(ea_l, src3, dst3, h_s)


def kernel(x, edge_index, edge_attr, batch, params):
    p = params
    src = edge_index[0]
    dst = edge_index[1]

    order = jnp.argsort(dst)
    nch = E_PAD // CHUNK
    ea_p = jnp.pad(edge_attr[order], ((0, E_PAD - E), (0, 0)))
    ea_p = ea_p.reshape(CHUNK, nch, DE).transpose(1, 0, 2).reshape(E_PAD, DE)
    srcp = jnp.pad(src[order], (0, E_PAD - E), constant_values=N)
    srcp = srcp.reshape(CHUNK, nch).T.reshape(-1)
    dstp = jnp.pad(dst[order], (0, E_PAD - E), constant_values=0)
    dstp = dstp.reshape(CHUNK, nch).T.reshape(-1)
    own0 = dstp < NHALF
    s0 = jnp.where(own0, srcp, N)
    d0 = jnp.where(own0, dstp, 0)
    s1 = jnp.where(own0, N, srcp)
    d1 = jnp.where(own0, 0, dstp - NHALF)
    src3 = jnp.stack([s0, s1]).reshape(NC, NS, CPT, CHUNK)
    dst3 = jnp.stack([d0, d1]).reshape(NC, NS, CPT, CHUNK)
    batch_t = batch.reshape(1, N)

    we, be = p["We"], p["be"]
    be3 = be[:, None, :]
    s1, s2 = _ea_stats(ea_p, we)
    mean_y = s1 / E + be3
    ey2 = s2 / E + 2.0 * be3 * (s1 / E) + be3 * be3
    var_y = ey2 - mean_y * mean_y
    sig_aff = jnp.sqrt(var_y + BN_EPS)
    g_aff = jnp.broadcast_to(p["ge"][:, None, :], sig_aff.shape)
    off_aff = be3 - mean_y
    t_aff = jnp.broadcast_to(p["bbe"][:, None, :], off_aff.shape)

    eas = _edge_embeddings(ea_p, we, off_aff, sig_aff, g_aff, t_aff)
    eas = [e.reshape(NS, PW, NHID) for e in eas]

    h_s = _input_encoder(x, p["Win"], p["bin"].reshape(1, NHID),
                         p["gin"].reshape(1, NHID), p["bbin"].reshape(1, NHID))

    for l in range(NLAYER):
        agg = _sc_scatter(eas[l], src3, dst3, h_s)
        h_s = _node_update(h_s, agg, p["Wc"][l],
                           p["gn"][l].reshape(1, NHID),
                           p["bbn"][l].reshape(1, NHID))

    return _pool_readout(
        h_s, batch_t,
        p["W1"], p["b1"].reshape(1, NHID), p["g1"].reshape(1, NHID),
        p["bb1"].reshape(1, NHID),
        p["W2"], p["b2"].reshape(1, NOUT), p["g2"].reshape(1, NOUT),
        p["bb2"].reshape(1, NOUT))


# speedup vs baseline: 15.1875x; 15.1875x over previous
import functools

import jax
import jax.numpy as jnp
from jax import lax
from jax.experimental import pallas as pl
from jax.experimental.pallas import tpu as pltpu
from jax.experimental.pallas import tpu_sc as plsc

N = 10000
E = 320000
DF = 128
DE = 16
NHID = 128
NOUT = 128
NLAYER = 4
NGRAPH = 64
BN_EPS = 1e-5

NC = 2
NS = 16
CHUNK = 128
CPT = 80
IDXH = CPT // 2
PW = CPT * CHUNK
E_PAD = NC * NS * PW
SENT = float(-1e9)

f32 = jnp.float32


def _dot16(a, w):
    return jnp.dot(a.astype(jnp.bfloat16), w.astype(jnp.bfloat16),
                   preferred_element_type=f32)


def _bn_cols(y, g, b):
    mu = jnp.mean(y, axis=0, keepdims=True)
    var = jnp.mean((y - mu) ** 2, axis=0, keepdims=True)
    return (y - mu) / jnp.sqrt(var + BN_EPS) * g + b


def _ea_stats_body(ea_ref, w_ref, s1_ref, s2_ref, acc1, acc2):
    j = pl.program_id(0)

    @pl.when(j == 0)
    def _():
        acc1[...] = jnp.zeros_like(acc1)
        acc2[...] = jnp.zeros_like(acc2)

    a = ea_ref[...]
    for l in range(NLAYER):
        d = _dot16(a, w_ref[l])
        acc1[l, ...] += jnp.sum(d, axis=0, keepdims=True)
        acc2[l, ...] += jnp.sum(d * d, axis=0, keepdims=True)

    @pl.when(j == pl.num_programs(0) - 1)
    def _():
        s1_ref[...] = acc1[...]
        s2_ref[...] = acc2[...]


_EA_BLK = 2048


def _ea_stats(ea_p, we):
    grid = E_PAD // _EA_BLK
    return pl.pallas_call(
        _ea_stats_body,
        grid=(grid,),
        in_specs=[pl.BlockSpec((_EA_BLK, DE), lambda j: (j, 0)),
                  pl.BlockSpec((NLAYER, DE, NHID), lambda j: (0, 0, 0))],
        out_specs=[pl.BlockSpec((NLAYER, 1, NHID), lambda j: (0, 0, 0)),
                   pl.BlockSpec((NLAYER, 1, NHID), lambda j: (0, 0, 0))],
        out_shape=[jax.ShapeDtypeStruct((NLAYER, 1, NHID), f32),
                   jax.ShapeDtypeStruct((NLAYER, 1, NHID), f32)],
        scratch_shapes=[pltpu.VMEM((NLAYER, 1, NHID), f32),
                        pltpu.VMEM((NLAYER, 1, NHID), f32)],
    )(ea_p, we)


def _ea_body(ea_ref, w_ref, off_ref, sig_ref, g_ref, t_ref, *out_refs):
    a = ea_ref[...]
    for l in range(NLAYER):
        y = (_dot16(a, w_ref[l]) + off_ref[l]) / sig_ref[l] * g_ref[l] + t_ref[l]
        out_refs[l][...] = jnp.maximum(y, 0.0)


_EA_BLK = 2048


def _edge_embeddings(ea_p, we, off, sig, g, t):
    grid = E_PAD // _EA_BLK
    return pl.pallas_call(
        _ea_body,
        grid=(grid,),
        in_specs=[pl.BlockSpec((_EA_BLK, DE), lambda j: (j, 0)),
                  pl.BlockSpec((NLAYER, DE, NHID), lambda j: (0, 0, 0)),
                  pl.BlockSpec((NLAYER, 1, NHID), lambda j: (0, 0, 0)),
                  pl.BlockSpec((NLAYER, 1, NHID), lambda j: (0, 0, 0)),
                  pl.BlockSpec((NLAYER, 1, NHID), lambda j: (0, 0, 0)),
                  pl.BlockSpec((NLAYER, 1, NHID), lambda j: (0, 0, 0))],
        out_specs=[pl.BlockSpec((_EA_BLK, NHID), lambda j: (j, 0))
                   for _ in range(NLAYER)],
        out_shape=[jax.ShapeDtypeStruct((E_PAD, NHID), f32)
                   for _ in range(NLAYER)],
    )(ea_p, we, off, sig, g, t)


def _enc_body(x_ref, w_ref, b_ref, g_ref, bb_ref, out_ref):
    y = _dot16(x_ref[...], w_ref[...]) + b_ref[...]
    out_ref[0:N, :] = jnp.maximum(_bn_cols(y, g_ref[...], bb_ref[...]), 0.0)
    out_ref[N:N + 8, :] = jnp.full((8, NHID), SENT, f32)


def _input_encoder(x, w, b, g, bb):
    return pl.pallas_call(
        _enc_body,
        out_shape=jax.ShapeDtypeStruct((N + 8, NHID), f32),
    )(x, w, b, g, bb)


def _node_body(h_ref, parts_ref, w_ref, g_ref, bb_ref, out_ref):
    h = h_ref[0:N, :]
    a = h + (parts_ref[0] + parts_ref[1])
    y = _dot16(a, w_ref[...])
    out_ref[0:N, :] = jnp.maximum(_bn_cols(y, g_ref[...], bb_ref[...]), 0.0) + h
    out_ref[N:N + 8, :] = jnp.full((8, NHID), SENT, f32)


def _node_update(h_s, parts, w, g, bb):
    return pl.pallas_call(
        _node_body,
        out_shape=jax.ShapeDtypeStruct((N + 8, NHID), f32),
    )(h_s, parts, w, g, bb)


def _pool_body(h_ref, bt_ref, w1, b1, g1, bb1, w2, b2, g2, bb2, out_ref):
    gid = lax.broadcasted_iota(jnp.int32, (NGRAPH, N), 0)
    onehot = (gid == bt_ref[...]).astype(f32)
    pooled = jnp.dot(onehot, h_ref[0:N, :], preferred_element_type=f32, precision=lax.Precision.HIGHEST)
    o = _dot16(pooled, w1[...]) + b1[...]
    o = jnp.maximum(_bn_cols(o, g1[...], bb1[...]), 0.0)
    o = _dot16(o, w2[...]) + b2[...]
    out_ref[...] = _bn_cols(o, g2[...], bb2[...])


def _pool_readout(h_s, batch_t, w1, b1, g1, bb1, w2, b2, g2, bb2):
    return pl.pallas_call(
        _pool_body,
        out_shape=jax.ShapeDtypeStruct((NGRAPH, NOUT), f32),
    )(h_s, batch_t, w1, b1, g1, bb1, w2, b2, g2, bb2)


def _sc_body(ea_hbm, src_hbm, dst_hbm, h_hbm, out_hbm,
             ea_v, g_v, src_v, dst_v, agg_sh, gsem):
    c = lax.axis_index("c")
    s = lax.axis_index("s")

    zero16 = jnp.zeros((16,), f32)

    def _zrow(i, carry):
        for cc in range(NHID // 16):
            g_v[i, pl.ds(cc * 16, 16)] = zero16
        return carry

    lax.fori_loop(0, CHUNK, _zrow, 0)

    base = s * 624

    @pl.when(s < NS - 1)
    def _():
        for k in range(4):
            pltpu.sync_copy(g_v, agg_sh.at[pl.ds(base + k * CHUNK, CHUNK)])
        pltpu.sync_copy(g_v.at[pl.ds(0, 112)],
                        agg_sh.at[pl.ds(base + 4 * CHUNK, 112)])

    @pl.when(s == NS - 1)
    def _():
        for k in range(5):
            pltpu.sync_copy(g_v, agg_sh.at[pl.ds(base + k * CHUNK, CHUNK)])

    plsc.subcore_barrier()

    for half in range(2):
        pltpu.sync_copy(src_hbm.at[c, s, pl.ds(half * IDXH, IDXH)], src_v)
        pltpu.sync_copy(dst_hbm.at[c, s, pl.ds(half * IDXH, IDXH)], dst_v)

        def _chunk(j, carry):
            jj = half * IDXH + j
            pltpu.sync_copy(ea_hbm.at[c, s, pl.ds(jj * CHUNK, CHUNK)], ea_v)
            pltpu.async_copy(h_hbm.at[src_v.at[j]], g_v, gsem).wait()

            def _crow(i, cc2):
                for cc in range(NHID // 16):
                    sl = pl.ds(cc * 16, 16)
                    g_v[i, sl] = jnp.maximum(g_v[i, sl] + ea_v[i, sl], 0.0)
                return cc2

            lax.fori_loop(0, CHUNK, _crow, 0)
            pltpu.sync_copy(g_v, agg_sh.at[dst_v.at[j]], add=True)
            return carry

        lax.fori_loop(0, IDXH, _chunk, 0)
    plsc.subcore_barrier()

    @pl.when(s < NS - 1)
    def _():
        pltpu.sync_copy(agg_sh.at[pl.ds(base, 624)],
                        out_hbm.at[c, pl.ds(base, 624)])

    @pl.when(s == NS - 1)
    def _():
        pltpu.sync_copy(agg_sh.at[pl.ds(base, 640)],
                        out_hbm.at[c, pl.ds(base, 640)])


@functools.lru_cache(maxsize=1)
def _sc_scatter_kernel():
    return pl.kernel(
        _sc_body,
        out_type=jax.ShapeDtypeStruct((NC, N, NHID), f32),
        mesh=plsc.VectorSubcoreMesh(core_axis_name="c", subcore_axis_name="s",
                                    num_cores=NC, num_subcores=NS),
        scratch_types=[
            pltpu.VMEM((CHUNK, NHID), f32),
            pltpu.VMEM((CHUNK, NHID), f32),
            pltpu.VMEM((IDXH, CHUNK), jnp.int32),
            pltpu.VMEM((IDXH, CHUNK), jnp.int32),
            pltpu.VMEM_SHARED((N, NHID), f32),
            pltpu.SemaphoreType.DMA,
        ],
    )


def _sc_scatter(ea_l, src3, dst3, h_s):
    return _sc_scatter_kernel()(ea_l, src3, dst3, h_s)


def kernel(x, edge_index, edge_attr, batch, params):
    p = params
    src = edge_index[0]
    dst = edge_index[1]

    ea_p = jnp.pad(edge_attr, ((0, E_PAD - E), (0, 0)))
    src3 = jnp.pad(src, (0, E_PAD - E), constant_values=N).reshape(
        NC, NS, CPT, CHUNK)
    dst3 = jnp.pad(dst, (0, E_PAD - E), constant_values=0).reshape(
        NC, NS, CPT, CHUNK)
    batch_t = batch.reshape(1, N)

    we, be = p["We"], p["be"]
    be3 = be[:, None, :]
    s1, s2 = _ea_stats(ea_p, we)
    mean_y = s1 / E + be3
    ey2 = s2 / E + 2.0 * be3 * (s1 / E) + be3 * be3
    var_y = ey2 - mean_y * mean_y
    sig_aff = jnp.sqrt(var_y + BN_EPS)
    g_aff = jnp.broadcast_to(p["ge"][:, None, :], sig_aff.shape)
    off_aff = be3 - mean_y
    t_aff = jnp.broadcast_to(p["bbe"][:, None, :], off_aff.shape)

    eas = _edge_embeddings(ea_p, we, off_aff, sig_aff, g_aff, t_aff)
    eas = [e.reshape(NC, NS, PW, NHID) for e in eas]

    h_s = _input_encoder(x, p["Win"], p["bin"].reshape(1, NHID),
                         p["gin"].reshape(1, NHID), p["bbin"].reshape(1, NHID))

    for l in range(NLAYER):
        parts = _sc_scatter(eas[l], src3, dst3, h_s)
        h_s = _node_update(h_s, parts, p["Wc"][l],
                           p["gn"][l].reshape(1, NHID),
                           p["bbn"][l].reshape(1, NHID))

    return _pool_readout(
        h_s, batch_t,
        p["W1"], p["b1"].reshape(1, NHID), p["g1"].reshape(1, NHID),
        p["bb1"].reshape(1, NHID),
        p["W2"], p["b2"].reshape(1, NOUT), p["g2"].reshape(1, NOUT),
        p["bb2"].reshape(1, NOUT))
